# Initial kernel scaffold; baseline (speedup 1.0000x reference)
import jax
import jax.numpy as jnp
from jax.experimental import pallas as pl


def kernel(x, g_attn, wq, wk, wv, wo, g_mlp, w_router, w1, b1, w2, b2):
    raise NotImplementedError("write your pallas kernel here")


# speedup vs baseline: 1.1633x; 1.1633x over previous
import jax
import jax.numpy as jnp
from jax.experimental import pallas as pl
from jax.experimental.pallas import tpu as pltpu

B, S, D = 1, 2048, 768
H, Dh = 12, 64
E, K = 8, 2
DFF = 1536
EPS = 1e-6
T = B * S
BLK = 256
QBLK = 512
HP = H // 2

f32 = jnp.float32
bf16 = jnp.bfloat16


def _qkv_kernel(x_ref, g_ref, w_ref, qkv_ref):
    x = x_ref[...]
    ms = jnp.mean(x * x, axis=-1, keepdims=True)
    h = (x * jax.lax.rsqrt(ms + EPS) * g_ref[...]).astype(bf16)
    qkv_ref[...] = jnp.dot(h, w_ref[...], preferred_element_type=f32).astype(bf16)


def _attn_kernel(q_ref, k_ref, v_ref, o_ref):
    i = pl.program_id(1)
    rows = jax.lax.broadcasted_iota(jnp.int32, (QBLK, S), 0) + i * QBLK
    cols = jax.lax.broadcasted_iota(jnp.int32, (QBLK, S), 1)
    causal = cols <= rows
    scale = jnp.float32(1.0) / jnp.sqrt(jnp.float32(Dh))
    for sub in range(2):
        q = q_ref[:, sub * Dh:(sub + 1) * Dh]
        k = k_ref[:, sub * Dh:(sub + 1) * Dh]
        v = v_ref[:, sub * Dh:(sub + 1) * Dh]
        s = jax.lax.dot_general(q, k, (((1,), (1,)), ((), ())),
                                preferred_element_type=f32) * scale
        s = jnp.where(causal, s, jnp.float32(-1e9))
        m = jnp.max(s, axis=-1, keepdims=True)
        p = jnp.exp(s - m)
        p = p / jnp.sum(p, axis=-1, keepdims=True)
        o = jnp.dot(p.astype(bf16), v, preferred_element_type=f32)
        o_ref[:, sub * Dh:(sub + 1) * Dh] = o.astype(bf16)


def _router_kernel(x_ref, a_ref, wo_ref, g_ref, wr_ref,
                   x2_ref, h2_ref, sc_ref, idx_ref, cnt_ref, sp_ref, loss_ref):
    i = pl.program_id(0)
    a = jnp.dot(a_ref[...], wo_ref[...], preferred_element_type=f32)
    x2 = x_ref[...] + a
    x2_ref[...] = x2
    ms = jnp.mean(x2 * x2, axis=-1, keepdims=True)
    h2 = x2 * jax.lax.rsqrt(ms + EPS) * g_ref[...]
    h2_ref[...] = h2.astype(bf16)
    logits = jnp.dot(h2, wr_ref[...], preferred_element_type=f32)
    m = jnp.max(logits, axis=-1, keepdims=True)
    ex = jnp.exp(logits - m)
    probs = ex / jnp.sum(ex, axis=-1, keepdims=True)
    ecols = jax.lax.broadcasted_iota(jnp.int32, (BLK, E), 1)
    i1 = jnp.argmax(probs, axis=-1).astype(jnp.int32)
    p1 = jnp.max(probs, axis=-1, keepdims=True)
    masked = jnp.where(ecols == i1[:, None], jnp.float32(-1.0), probs)
    i2 = jnp.argmax(masked, axis=-1).astype(jnp.int32)
    p2 = jnp.max(masked, axis=-1, keepdims=True)
    sc_ref[...] = jnp.concatenate([p1, p2], axis=1)
    idx_ref[...] = jnp.concatenate([i1[:, None], i2[:, None]], axis=1)
    onehot = ((ecols == i1[:, None]).astype(f32)
              + (ecols == i2[:, None]).astype(f32))
    cnt_part = jnp.sum(onehot, axis=0, keepdims=True)
    sp_part = jnp.sum(probs, axis=0, keepdims=True)

    @pl.when(i == 0)
    def _():
        cnt_ref[...] = jnp.zeros_like(cnt_ref)
        sp_ref[...] = jnp.zeros_like(sp_ref)

    cnt_ref[...] += cnt_part
    sp_ref[...] += sp_part

    @pl.when(i == pl.num_programs(0) - 1)
    def _():
        density = cnt_ref[...] / jnp.float32(T * K)
        meanp = sp_ref[...] / jnp.float32(T)
        loss_ref[...] = jnp.float32(E) * jnp.sum(density * meanp).reshape(1, 1)


def _moe_kernel(h2_ref, w1_ref, b1_ref, w2_ref, b2_ref, idx_ref, eo_ref):
    e = pl.program_id(0)

    @pl.when(e == 0)
    def _():
        eo_ref[...] = jnp.zeros_like(eo_ref)

    for half in range(2):
        lo, hi = half * (T // 2), (half + 1) * (T // 2)
        h2 = h2_ref[lo:hi, :]
        hh = jnp.dot(h2, w1_ref[0], preferred_element_type=f32) + b1_ref[0]
        hh = jax.nn.gelu(hh).astype(bf16)
        oute = jnp.dot(hh, w2_ref[0], preferred_element_type=f32) + b2_ref[0]
        m0 = (idx_ref[lo:hi, 0:1] == e).astype(f32)
        m1 = (idx_ref[lo:hi, 1:2] == e).astype(f32)
        eo_ref[lo:hi, 0, :] += m0 * oute
        eo_ref[lo:hi, 1, :] += m1 * oute


def _combine_kernel(x2_ref, eoA_ref, eoB_ref, sc_ref, o_ref):
    sc = sc_ref[...]
    o_ref[...] = (x2_ref[...] + sc[:, 0:1] * eoA_ref[...]
                  + sc[:, 1:2] * eoB_ref[...])


def kernel(x, g_attn, wq, wk, wv, wo, g_mlp, w_router, w1, b1, w2, b2):
    x2d = x.reshape(T, D)
    wqkv = jnp.concatenate([wq, wk, wv], axis=1).astype(bf16)

    qkv = pl.pallas_call(
        _qkv_kernel,
        grid=(T // BLK,),
        in_specs=[
            pl.BlockSpec((BLK, D), lambda i: (i, 0)),
            pl.BlockSpec((1, D), lambda i: (0, 0)),
            pl.BlockSpec((D, 3 * D), lambda i: (0, 0)),
        ],
        out_specs=pl.BlockSpec((BLK, 3 * D), lambda i: (i, 0)),
        out_shape=jax.ShapeDtypeStruct((T, 3 * D), bf16),
    )(x2d, g_attn.reshape(1, D), wqkv)

    attn = pl.pallas_call(
        _attn_kernel,
        grid=(HP, S // QBLK),
        in_specs=[
            pl.BlockSpec((QBLK, 2 * Dh), lambda h, i: (i, h)),
            pl.BlockSpec((S, 2 * Dh), lambda h, i: (0, HP + h)),
            pl.BlockSpec((S, 2 * Dh), lambda h, i: (0, 2 * HP + h)),
        ],
        out_specs=pl.BlockSpec((QBLK, 2 * Dh), lambda h, i: (i, h)),
        out_shape=jax.ShapeDtypeStruct((T, D), bf16),
    )(qkv, qkv, qkv)

    x2, h2, sc, idx, cnt, sp, loss = pl.pallas_call(
        _router_kernel,
        grid=(T // BLK,),
        in_specs=[
            pl.BlockSpec((BLK, D), lambda i: (i, 0)),
            pl.BlockSpec((BLK, D), lambda i: (i, 0)),
            pl.BlockSpec((D, D), lambda i: (0, 0)),
            pl.BlockSpec((1, D), lambda i: (0, 0)),
            pl.BlockSpec((D, E), lambda i: (0, 0)),
        ],
        out_specs=[
            pl.BlockSpec((BLK, D), lambda i: (i, 0)),
            pl.BlockSpec((BLK, D), lambda i: (i, 0)),
            pl.BlockSpec((BLK, K), lambda i: (i, 0)),
            pl.BlockSpec((BLK, K), lambda i: (i, 0)),
            pl.BlockSpec((1, E), lambda i: (0, 0)),
            pl.BlockSpec((1, E), lambda i: (0, 0)),
            pl.BlockSpec((1, 1), lambda i: (0, 0)),
        ],
        out_shape=[
            jax.ShapeDtypeStruct((T, D), f32),
            jax.ShapeDtypeStruct((T, D), bf16),
            jax.ShapeDtypeStruct((T, K), f32),
            jax.ShapeDtypeStruct((T, K), jnp.int32),
            jax.ShapeDtypeStruct((1, E), f32),
            jax.ShapeDtypeStruct((1, E), f32),
            jax.ShapeDtypeStruct((1, 1), f32),
        ],
    )(x2d, attn, wo.astype(bf16), g_mlp.reshape(1, D), w_router)

    eo = pl.pallas_call(
        _moe_kernel,
        grid=(E,),
        in_specs=[
            pl.BlockSpec((T, D), lambda e: (0, 0)),
            pl.BlockSpec((1, D, DFF), lambda e: (e, 0, 0)),
            pl.BlockSpec((1, 1, DFF), lambda e: (e, 0, 0)),
            pl.BlockSpec((1, DFF, D), lambda e: (e, 0, 0)),
            pl.BlockSpec((1, 1, D), lambda e: (e, 0, 0)),
            pl.BlockSpec((T, K), lambda e: (0, 0)),
        ],
        out_specs=pl.BlockSpec((T, K, D), lambda e: (0, 0, 0)),
        out_shape=jax.ShapeDtypeStruct((T, K, D), f32),
        compiler_params=pltpu.CompilerParams(
            vmem_limit_bytes=100 * 1024 * 1024),
    )(h2, w1.astype(bf16), b1.reshape(E, 1, DFF), w2.astype(bf16),
      b2.reshape(E, 1, D), idx)

    eoA = eo[:T // 2].reshape(T, D)
    eoB = eo[T // 2:].reshape(T, D)
    final = pl.pallas_call(
        _combine_kernel,
        grid=(T // BLK,),
        in_specs=[
            pl.BlockSpec((BLK, D), lambda i: (i, 0)),
            pl.BlockSpec((BLK, D), lambda i: (i, 0)),
            pl.BlockSpec((BLK, D), lambda i: (i, 0)),
            pl.BlockSpec((BLK, K), lambda i: (i, 0)),
        ],
        out_specs=pl.BlockSpec((BLK, D), lambda i: (i, 0)),
        out_shape=jax.ShapeDtypeStruct((T, D), f32),
    )(x2, eoA, eoB, sc)

    return final.reshape(B, S, D), loss.reshape(()), cnt.reshape(E)


# speedup vs baseline: 1.2241x; 1.0523x over previous
import jax
import jax.numpy as jnp
from jax.experimental import pallas as pl
from jax.experimental.pallas import tpu as pltpu

B, S, D = 1, 2048, 768
H, Dh = 12, 64
E, K = 8, 2
DFF = 1536
EPS = 1e-6
T = B * S
BLK = 256
QBLK = 512
HP = H // 2

f32 = jnp.float32
bf16 = jnp.bfloat16


def _qkv_kernel(x_ref, g_ref, w_ref, qkv_ref):
    x = x_ref[...]
    ms = jnp.mean(x * x, axis=-1, keepdims=True)
    h = (x * jax.lax.rsqrt(ms + EPS) * g_ref[...]).astype(bf16)
    qkv_ref[...] = jnp.dot(h, w_ref[...], preferred_element_type=f32).astype(bf16)


def _attn_kernel(q_ref, k_ref, v_ref, o_ref, m_ref, l_ref, oacc_ref):
    i = pl.program_id(1)
    rows = jax.lax.broadcasted_iota(jnp.int32, (QBLK, QBLK), 0) + i * QBLK
    cols0 = jax.lax.broadcasted_iota(jnp.int32, (QBLK, QBLK), 1)
    scale = jnp.float32(1.0) / jnp.sqrt(jnp.float32(Dh))
    nch = S // QBLK
    for sub in range(2):
        cs = slice(sub * Dh, (sub + 1) * Dh)
        q = q_ref[:, cs]
        m_ref[...] = jnp.full((QBLK, 1), -1e30, f32)
        l_ref[...] = jnp.zeros((QBLK, 1), f32)
        oacc_ref[...] = jnp.zeros((QBLK, Dh), f32)
        for j in range(nch):
            @pl.when(j <= i)
            def _():
                kc = k_ref[j * QBLK:(j + 1) * QBLK, cs]
                vc = v_ref[j * QBLK:(j + 1) * QBLK, cs]
                s = jax.lax.dot_general(q, kc, (((1,), (1,)), ((), ())),
                                        preferred_element_type=f32) * scale
                s = jnp.where(cols0 + j * QBLK <= rows, s, jnp.float32(-1e9))
                mj = jnp.max(s, axis=-1, keepdims=True)
                mnew = jnp.maximum(m_ref[...], mj)
                corr = jnp.exp(m_ref[...] - mnew)
                p = jnp.exp(s - mnew)
                l_ref[...] = l_ref[...] * corr + jnp.sum(p, axis=-1,
                                                         keepdims=True)
                oacc_ref[...] = (oacc_ref[...] * corr
                                 + jnp.dot(p.astype(bf16), vc,
                                           preferred_element_type=f32))
                m_ref[...] = mnew
        o = oacc_ref[...] * (jnp.float32(1.0) / l_ref[...])
        o_ref[:, cs] = o.astype(bf16)


def _router_kernel(x_ref, a_ref, wo_ref, g_ref, wr_ref,
                   x2_ref, h2_ref, sc_ref, idx_ref, cnt_ref, sp_ref, loss_ref):
    i = pl.program_id(0)
    a = jnp.dot(a_ref[...], wo_ref[...], preferred_element_type=f32)
    x2 = x_ref[...] + a
    x2_ref[...] = x2
    ms = jnp.mean(x2 * x2, axis=-1, keepdims=True)
    h2 = x2 * jax.lax.rsqrt(ms + EPS) * g_ref[...]
    h2_ref[...] = h2.astype(bf16)
    logits = jnp.dot(h2, wr_ref[...], preferred_element_type=f32)
    m = jnp.max(logits, axis=-1, keepdims=True)
    ex = jnp.exp(logits - m)
    probs = ex / jnp.sum(ex, axis=-1, keepdims=True)
    ecols = jax.lax.broadcasted_iota(jnp.int32, (BLK, E), 1)
    i1 = jnp.argmax(probs, axis=-1).astype(jnp.int32)
    p1 = jnp.max(probs, axis=-1, keepdims=True)
    masked = jnp.where(ecols == i1[:, None], jnp.float32(-1.0), probs)
    i2 = jnp.argmax(masked, axis=-1).astype(jnp.int32)
    p2 = jnp.max(masked, axis=-1, keepdims=True)
    sc_ref[...] = jnp.concatenate([p1, p2], axis=1)
    idx_ref[...] = jnp.concatenate([i1[:, None], i2[:, None]], axis=1)
    onehot = ((ecols == i1[:, None]).astype(f32)
              + (ecols == i2[:, None]).astype(f32))
    cnt_part = jnp.sum(onehot, axis=0, keepdims=True)
    sp_part = jnp.sum(probs, axis=0, keepdims=True)

    @pl.when(i == 0)
    def _():
        cnt_ref[...] = jnp.zeros_like(cnt_ref)
        sp_ref[...] = jnp.zeros_like(sp_ref)

    cnt_ref[...] += cnt_part
    sp_ref[...] += sp_part

    @pl.when(i == pl.num_programs(0) - 1)
    def _():
        density = cnt_ref[...] / jnp.float32(T * K)
        meanp = sp_ref[...] / jnp.float32(T)
        loss_ref[...] = jnp.float32(E) * jnp.sum(density * meanp).reshape(1, 1)


def _moe_kernel(h2_ref, w1_ref, b1_ref, w2_ref, b2_ref, idx_ref, eo_ref):
    e = pl.program_id(0)

    @pl.when(e == 0)
    def _():
        eo_ref[...] = jnp.zeros_like(eo_ref)

    for half in range(2):
        lo, hi = half * (T // 2), (half + 1) * (T // 2)
        h2 = h2_ref[lo:hi, :]
        hh = jnp.dot(h2, w1_ref[0], preferred_element_type=f32) + b1_ref[0]
        hh = jax.nn.gelu(hh).astype(bf16)
        oute = jnp.dot(hh, w2_ref[0], preferred_element_type=f32) + b2_ref[0]
        m0 = (idx_ref[lo:hi, 0:1] == e).astype(f32)
        m1 = (idx_ref[lo:hi, 1:2] == e).astype(f32)
        eo_ref[0, lo:hi, :] += m0 * oute
        eo_ref[1, lo:hi, :] += m1 * oute


def _combine_kernel(x2_ref, eoA_ref, eoB_ref, sc_ref, o_ref):
    sc = sc_ref[...]
    a = jnp.concatenate([eoA_ref[0][:, None, :], eoA_ref[1][:, None, :]],
                        axis=1).reshape(BLK, D)
    b = jnp.concatenate([eoB_ref[0][:, None, :], eoB_ref[1][:, None, :]],
                        axis=1).reshape(BLK, D)
    o_ref[...] = x2_ref[...] + sc[:, 0:1] * a + sc[:, 1:2] * b


def kernel(x, g_attn, wq, wk, wv, wo, g_mlp, w_router, w1, b1, w2, b2):
    x2d = x.reshape(T, D)
    wqkv = jnp.concatenate([wq, wk, wv], axis=1).astype(bf16)

    qkv = pl.pallas_call(
        _qkv_kernel,
        grid=(T // BLK,),
        in_specs=[
            pl.BlockSpec((BLK, D), lambda i: (i, 0)),
            pl.BlockSpec((1, D), lambda i: (0, 0)),
            pl.BlockSpec((D, 3 * D), lambda i: (0, 0)),
        ],
        out_specs=pl.BlockSpec((BLK, 3 * D), lambda i: (i, 0)),
        out_shape=jax.ShapeDtypeStruct((T, 3 * D), bf16),
    )(x2d, g_attn.reshape(1, D), wqkv)

    attn = pl.pallas_call(
        _attn_kernel,
        grid=(HP, S // QBLK),
        in_specs=[
            pl.BlockSpec((QBLK, 2 * Dh), lambda h, i: (i, h)),
            pl.BlockSpec((S, 2 * Dh), lambda h, i: (0, HP + h)),
            pl.BlockSpec((S, 2 * Dh), lambda h, i: (0, 2 * HP + h)),
        ],
        out_specs=pl.BlockSpec((QBLK, 2 * Dh), lambda h, i: (i, h)),
        out_shape=jax.ShapeDtypeStruct((T, D), bf16),
        scratch_shapes=[
            pltpu.VMEM((QBLK, 1), f32),
            pltpu.VMEM((QBLK, 1), f32),
            pltpu.VMEM((QBLK, Dh), f32),
        ],
    )(qkv, qkv, qkv)

    x2, h2, sc, idx, cnt, sp, loss = pl.pallas_call(
        _router_kernel,
        grid=(T // BLK,),
        in_specs=[
            pl.BlockSpec((BLK, D), lambda i: (i, 0)),
            pl.BlockSpec((BLK, D), lambda i: (i, 0)),
            pl.BlockSpec((D, D), lambda i: (0, 0)),
            pl.BlockSpec((1, D), lambda i: (0, 0)),
            pl.BlockSpec((D, E), lambda i: (0, 0)),
        ],
        out_specs=[
            pl.BlockSpec((BLK, D), lambda i: (i, 0)),
            pl.BlockSpec((BLK, D), lambda i: (i, 0)),
            pl.BlockSpec((BLK, K), lambda i: (i, 0)),
            pl.BlockSpec((BLK, K), lambda i: (i, 0)),
            pl.BlockSpec((1, E), lambda i: (0, 0)),
            pl.BlockSpec((1, E), lambda i: (0, 0)),
            pl.BlockSpec((1, 1), lambda i: (0, 0)),
        ],
        out_shape=[
            jax.ShapeDtypeStruct((T, D), f32),
            jax.ShapeDtypeStruct((T, D), bf16),
            jax.ShapeDtypeStruct((T, K), f32),
            jax.ShapeDtypeStruct((T, K), jnp.int32),
            jax.ShapeDtypeStruct((1, E), f32),
            jax.ShapeDtypeStruct((1, E), f32),
            jax.ShapeDtypeStruct((1, 1), f32),
        ],
    )(x2d, attn, wo.astype(bf16), g_mlp.reshape(1, D), w_router)

    eo = pl.pallas_call(
        _moe_kernel,
        grid=(E,),
        in_specs=[
            pl.BlockSpec((T, D), lambda e: (0, 0)),
            pl.BlockSpec((1, D, DFF), lambda e: (e, 0, 0)),
            pl.BlockSpec((1, 1, DFF), lambda e: (e, 0, 0)),
            pl.BlockSpec((1, DFF, D), lambda e: (e, 0, 0)),
            pl.BlockSpec((1, 1, D), lambda e: (e, 0, 0)),
            pl.BlockSpec((T, K), lambda e: (0, 0)),
        ],
        out_specs=pl.BlockSpec((K, T, D), lambda e: (0, 0, 0)),
        out_shape=jax.ShapeDtypeStruct((K, T, D), f32),
        compiler_params=pltpu.CompilerParams(
            vmem_limit_bytes=100 * 1024 * 1024),
    )(h2, w1.astype(bf16), b1.reshape(E, 1, DFF), w2.astype(bf16),
      b2.reshape(E, 1, D), idx)

    final = pl.pallas_call(
        _combine_kernel,
        grid=(T // BLK,),
        in_specs=[
            pl.BlockSpec((BLK, D), lambda i: (i, 0)),
            pl.BlockSpec((K, BLK // 2, D), lambda i: (0, i, 0)),
            pl.BlockSpec((K, BLK // 2, D), lambda i: (0, (T // BLK) + i, 0)),
            pl.BlockSpec((BLK, K), lambda i: (i, 0)),
        ],
        out_specs=pl.BlockSpec((BLK, D), lambda i: (i, 0)),
        out_shape=jax.ShapeDtypeStruct((T, D), f32),
    )(x2, eo, eo, sc)

    return final.reshape(B, S, D), loss.reshape(()), cnt.reshape(E)


# speedup vs baseline: 1.4846x; 1.2128x over previous
import jax
import jax.numpy as jnp
from jax.experimental import pallas as pl
from jax.experimental.pallas import tpu as pltpu

B, S, D = 1, 2048, 768
H, Dh = 12, 64
E, K = 8, 2
DFF = 1536
EPS = 1e-6
T = B * S
BLK = 256
QBLK = 512
HP = H // 2

f32 = jnp.float32
bf16 = jnp.bfloat16


def _qkv_kernel(x_ref, g_ref, w_ref, qkv_ref):
    x = x_ref[...]
    ms = jnp.mean(x * x, axis=-1, keepdims=True)
    h = (x * jax.lax.rsqrt(ms + EPS) * g_ref[...]).astype(bf16)
    qkv_ref[...] = jnp.dot(h, w_ref[...], preferred_element_type=f32).astype(bf16)


def _attn_kernel(q_ref, k_ref, v_ref, o_ref, m_ref, oacc_ref):
    i = pl.program_id(1)
    tri = (jax.lax.broadcasted_iota(jnp.int32, (QBLK, QBLK), 1)
           <= jax.lax.broadcasted_iota(jnp.int32, (QBLK, QBLK), 0))
    scale = jnp.float32(1.0) / jnp.sqrt(jnp.float32(Dh))
    ones = jnp.ones((QBLK, Dh), bf16)
    nch = S // QBLK
    for sub in range(2):
        cs = slice(sub * Dh, (sub + 1) * Dh)
        q = (q_ref[:, cs].astype(f32) * scale).astype(bf16)
        kc = k_ref[pl.ds(i * QBLK, QBLK), cs]
        vc = v_ref[pl.ds(i * QBLK, QBLK), cs]
        s = jax.lax.dot_general(q, kc, (((1,), (1,)), ((), ())),
                                preferred_element_type=f32).astype(bf16)
        s = jnp.where(tri, s, jnp.asarray(-1e9, bf16))
        mj = jnp.max(s, axis=-1, keepdims=True).astype(f32)
        m_ref[...] = mj
        p = jnp.exp(s - mj.astype(bf16))
        vext = jnp.concatenate([vc, ones], axis=1)
        oacc_ref[...] = jnp.dot(p, vext, preferred_element_type=f32)
        for j in range(nch - 1):
            @pl.when(j < i)
            def _():
                kc = k_ref[j * QBLK:(j + 1) * QBLK, cs]
                vc = v_ref[j * QBLK:(j + 1) * QBLK, cs]
                s = jax.lax.dot_general(q, kc, (((1,), (1,)), ((), ())),
                                        preferred_element_type=f32
                                        ).astype(bf16)
                mj = jnp.max(s, axis=-1, keepdims=True).astype(f32)
                mnew = jnp.maximum(m_ref[...], mj)
                corr = jnp.exp(m_ref[...] - mnew)
                p = jnp.exp(s - mnew.astype(bf16))
                vext = jnp.concatenate([vc, ones], axis=1)
                oacc_ref[...] = (oacc_ref[...] * corr
                                 + jnp.dot(p, vext,
                                           preferred_element_type=f32))
                m_ref[...] = mnew
        acc = oacc_ref[...]
        o = acc[:, :Dh] * (jnp.float32(1.0) / acc[:, Dh:Dh + 1])
        o_ref[:, cs] = o.astype(bf16)


def _router_kernel(x_ref, a_ref, wo_ref, g_ref, wr_ref,
                   x2_ref, h2_ref, sc_ref, idx_ref, cnt_ref, sp_ref, loss_ref):
    i = pl.program_id(0)
    a = jnp.dot(a_ref[...], wo_ref[...], preferred_element_type=f32)
    x2 = x_ref[...] + a
    x2_ref[...] = x2
    ms = jnp.mean(x2 * x2, axis=-1, keepdims=True)
    h2 = x2 * jax.lax.rsqrt(ms + EPS) * g_ref[...]
    h2_ref[...] = h2.astype(bf16)
    logits = jnp.dot(h2, wr_ref[...], preferred_element_type=f32)
    m = jnp.max(logits, axis=-1, keepdims=True)
    ex = jnp.exp(logits - m)
    probs = ex / jnp.sum(ex, axis=-1, keepdims=True)
    ecols = jax.lax.broadcasted_iota(jnp.int32, (BLK, E), 1)
    i1 = jnp.argmax(probs, axis=-1).astype(jnp.int32)
    p1 = jnp.max(probs, axis=-1, keepdims=True)
    masked = jnp.where(ecols == i1[:, None], jnp.float32(-1.0), probs)
    i2 = jnp.argmax(masked, axis=-1).astype(jnp.int32)
    p2 = jnp.max(masked, axis=-1, keepdims=True)
    sc_ref[...] = jnp.concatenate([p1, p2], axis=1)
    idx_ref[...] = jnp.concatenate([i1[:, None], i2[:, None]], axis=1)
    onehot = ((ecols == i1[:, None]).astype(f32)
              + (ecols == i2[:, None]).astype(f32))
    cnt_part = jnp.sum(onehot, axis=0, keepdims=True)
    sp_part = jnp.sum(probs, axis=0, keepdims=True)

    @pl.when(i == 0)
    def _():
        cnt_ref[...] = jnp.zeros_like(cnt_ref)
        sp_ref[...] = jnp.zeros_like(sp_ref)

    cnt_ref[...] += cnt_part
    sp_ref[...] += sp_part

    @pl.when(i == pl.num_programs(0) - 1)
    def _():
        density = cnt_ref[...] / jnp.float32(T * K)
        meanp = sp_ref[...] / jnp.float32(T)
        loss_ref[...] = jnp.float32(E) * jnp.sum(density * meanp).reshape(1, 1)


def _moe_kernel(h2_ref, w1_ref, b1_ref, w2_ref, b2_ref, idx_ref, eo_ref):
    e = pl.program_id(0)

    @pl.when(e == 0)
    def _():
        eo_ref[...] = jnp.zeros_like(eo_ref)

    for half in range(2):
        lo, hi = half * (T // 2), (half + 1) * (T // 2)
        h2 = h2_ref[lo:hi, :]
        hh = jnp.dot(h2, w1_ref[0], preferred_element_type=f32) + b1_ref[0]
        hh = jax.nn.gelu(hh).astype(bf16)
        oute = jnp.dot(hh, w2_ref[0], preferred_element_type=f32) + b2_ref[0]
        m0 = (idx_ref[lo:hi, 0:1] == e).astype(f32)
        m1 = (idx_ref[lo:hi, 1:2] == e).astype(f32)
        eo_ref[0, lo:hi, :] += m0 * oute
        eo_ref[1, lo:hi, :] += m1 * oute


def _combine_kernel(x2_ref, eoA_ref, eoB_ref, sc_ref, o_ref):
    sc = sc_ref[...]
    a = jnp.concatenate([eoA_ref[0][:, None, :], eoA_ref[1][:, None, :]],
                        axis=1).reshape(BLK, D)
    b = jnp.concatenate([eoB_ref[0][:, None, :], eoB_ref[1][:, None, :]],
                        axis=1).reshape(BLK, D)
    o_ref[...] = x2_ref[...] + sc[:, 0:1] * a + sc[:, 1:2] * b


def kernel(x, g_attn, wq, wk, wv, wo, g_mlp, w_router, w1, b1, w2, b2):
    x2d = x.reshape(T, D)
    wqkv = jnp.concatenate([wq, wk, wv], axis=1).astype(bf16)

    qkv = pl.pallas_call(
        _qkv_kernel,
        grid=(T // BLK,),
        in_specs=[
            pl.BlockSpec((BLK, D), lambda i: (i, 0)),
            pl.BlockSpec((1, D), lambda i: (0, 0)),
            pl.BlockSpec((D, 3 * D), lambda i: (0, 0)),
        ],
        out_specs=pl.BlockSpec((BLK, 3 * D), lambda i: (i, 0)),
        out_shape=jax.ShapeDtypeStruct((T, 3 * D), bf16),
    )(x2d, g_attn.reshape(1, D), wqkv)

    attn = pl.pallas_call(
        _attn_kernel,
        grid=(HP, S // QBLK),
        in_specs=[
            pl.BlockSpec((QBLK, 2 * Dh), lambda h, i: (i, h)),
            pl.BlockSpec((S, 2 * Dh), lambda h, i: (0, HP + h)),
            pl.BlockSpec((S, 2 * Dh), lambda h, i: (0, 2 * HP + h)),
        ],
        out_specs=pl.BlockSpec((QBLK, 2 * Dh), lambda h, i: (i, h)),
        out_shape=jax.ShapeDtypeStruct((T, D), bf16),
        scratch_shapes=[
            pltpu.VMEM((QBLK, 1), f32),
            pltpu.VMEM((QBLK, 2 * Dh), f32),
        ],
    )(qkv, qkv, qkv)

    x2, h2, sc, idx, cnt, sp, loss = pl.pallas_call(
        _router_kernel,
        grid=(T // BLK,),
        in_specs=[
            pl.BlockSpec((BLK, D), lambda i: (i, 0)),
            pl.BlockSpec((BLK, D), lambda i: (i, 0)),
            pl.BlockSpec((D, D), lambda i: (0, 0)),
            pl.BlockSpec((1, D), lambda i: (0, 0)),
            pl.BlockSpec((D, E), lambda i: (0, 0)),
        ],
        out_specs=[
            pl.BlockSpec((BLK, D), lambda i: (i, 0)),
            pl.BlockSpec((BLK, D), lambda i: (i, 0)),
            pl.BlockSpec((BLK, K), lambda i: (i, 0)),
            pl.BlockSpec((BLK, K), lambda i: (i, 0)),
            pl.BlockSpec((1, E), lambda i: (0, 0)),
            pl.BlockSpec((1, E), lambda i: (0, 0)),
            pl.BlockSpec((1, 1), lambda i: (0, 0)),
        ],
        out_shape=[
            jax.ShapeDtypeStruct((T, D), f32),
            jax.ShapeDtypeStruct((T, D), bf16),
            jax.ShapeDtypeStruct((T, K), f32),
            jax.ShapeDtypeStruct((T, K), jnp.int32),
            jax.ShapeDtypeStruct((1, E), f32),
            jax.ShapeDtypeStruct((1, E), f32),
            jax.ShapeDtypeStruct((1, 1), f32),
        ],
    )(x2d, attn, wo.astype(bf16), g_mlp.reshape(1, D), w_router)

    eo = pl.pallas_call(
        _moe_kernel,
        grid=(E,),
        in_specs=[
            pl.BlockSpec((T, D), lambda e: (0, 0)),
            pl.BlockSpec((1, D, DFF), lambda e: (e, 0, 0)),
            pl.BlockSpec((1, 1, DFF), lambda e: (e, 0, 0)),
            pl.BlockSpec((1, DFF, D), lambda e: (e, 0, 0)),
            pl.BlockSpec((1, 1, D), lambda e: (e, 0, 0)),
            pl.BlockSpec((T, K), lambda e: (0, 0)),
        ],
        out_specs=pl.BlockSpec((K, T, D), lambda e: (0, 0, 0)),
        out_shape=jax.ShapeDtypeStruct((K, T, D), f32),
        compiler_params=pltpu.CompilerParams(
            vmem_limit_bytes=100 * 1024 * 1024),
    )(h2, w1.astype(bf16), b1.reshape(E, 1, DFF), w2.astype(bf16),
      b2.reshape(E, 1, D), idx)

    final = pl.pallas_call(
        _combine_kernel,
        grid=(T // BLK,),
        in_specs=[
            pl.BlockSpec((BLK, D), lambda i: (i, 0)),
            pl.BlockSpec((K, BLK // 2, D), lambda i: (0, i, 0)),
            pl.BlockSpec((K, BLK // 2, D), lambda i: (0, (T // BLK) + i, 0)),
            pl.BlockSpec((BLK, K), lambda i: (i, 0)),
        ],
        out_specs=pl.BlockSpec((BLK, D), lambda i: (i, 0)),
        out_shape=jax.ShapeDtypeStruct((T, D), f32),
    )(x2, eo, eo, sc)

    return final.reshape(B, S, D), loss.reshape(()), cnt.reshape(E)


# speedup vs baseline: 1.4929x; 1.0055x over previous
import jax
import jax.numpy as jnp
from jax.experimental import pallas as pl
from jax.experimental.pallas import tpu as pltpu

B, S, D = 1, 2048, 768
H, Dh = 12, 64
E, K = 8, 2
DFF = 1536
EPS = 1e-6
T = B * S
BLK = 256
QBLK = 512
HP = H // 2

f32 = jnp.float32
bf16 = jnp.bfloat16


def _qkv_kernel(x_ref, g_ref, w_ref, qkv_ref):
    x = x_ref[...]
    ms = jnp.mean(x * x, axis=-1, keepdims=True)
    h = (x * jax.lax.rsqrt(ms + EPS) * g_ref[...]).astype(bf16)
    qkv_ref[...] = jnp.dot(h, w_ref[...], preferred_element_type=f32).astype(bf16)


def _attn_kernel(q_ref, k_ref, v_ref, o_ref, m_ref, oacc_ref):
    i = pl.program_id(1)
    tri = (jax.lax.broadcasted_iota(jnp.int32, (QBLK, QBLK), 1)
           <= jax.lax.broadcasted_iota(jnp.int32, (QBLK, QBLK), 0))
    scale = jnp.float32(1.0) / jnp.sqrt(jnp.float32(Dh))
    ones = jnp.ones((QBLK, Dh), bf16)
    nch = S // QBLK
    for sub in range(2):
        cs = slice(sub * Dh, (sub + 1) * Dh)
        q = (q_ref[:, cs].astype(f32) * scale).astype(bf16)
        kc = k_ref[pl.ds(i * QBLK, QBLK), cs]
        vc = v_ref[pl.ds(i * QBLK, QBLK), cs]
        s = jax.lax.dot_general(q, kc, (((1,), (1,)), ((), ())),
                                preferred_element_type=f32).astype(bf16)
        s = jnp.where(tri, s, jnp.asarray(-1e9, bf16))
        mj = jnp.max(s, axis=-1, keepdims=True).astype(f32)
        m_ref[...] = mj
        p = jnp.exp(s - mj.astype(bf16))
        vext = jnp.concatenate([vc, ones], axis=1)
        oacc_ref[...] = jnp.dot(p, vext, preferred_element_type=f32)
        for j in range(nch - 1):
            @pl.when(j < i)
            def _():
                kc = k_ref[j * QBLK:(j + 1) * QBLK, cs]
                vc = v_ref[j * QBLK:(j + 1) * QBLK, cs]
                s = jax.lax.dot_general(q, kc, (((1,), (1,)), ((), ())),
                                        preferred_element_type=f32
                                        ).astype(bf16)
                mj = jnp.max(s, axis=-1, keepdims=True).astype(f32)
                mnew = jnp.maximum(m_ref[...], mj)
                corr = jnp.exp(m_ref[...] - mnew)
                p = jnp.exp(s - mnew.astype(bf16))
                vext = jnp.concatenate([vc, ones], axis=1)
                oacc_ref[...] = (oacc_ref[...] * corr
                                 + jnp.dot(p, vext,
                                           preferred_element_type=f32))
                m_ref[...] = mnew
        acc = oacc_ref[...]
        o = acc[:, :Dh] * (jnp.float32(1.0) / acc[:, Dh:Dh + 1])
        o_ref[:, cs] = o.astype(bf16)


def _router_kernel(x_ref, a_ref, wo_ref, g_ref, wr_ref,
                   x2_ref, h2_ref, sc_ref, idx_ref, cnt_ref, sp_ref, loss_ref):
    i = pl.program_id(0)
    a = jnp.dot(a_ref[...], wo_ref[...], preferred_element_type=f32)
    x2 = x_ref[...] + a
    x2_ref[...] = x2
    ms = jnp.mean(x2 * x2, axis=-1, keepdims=True)
    h2 = x2 * jax.lax.rsqrt(ms + EPS) * g_ref[...]
    h2_ref[...] = h2.astype(bf16)
    logits = jnp.dot(h2, wr_ref[...], preferred_element_type=f32)
    m = jnp.max(logits, axis=-1, keepdims=True)
    ex = jnp.exp(logits - m)
    probs = ex / jnp.sum(ex, axis=-1, keepdims=True)
    ecols = jax.lax.broadcasted_iota(jnp.int32, (BLK, E), 1)
    i1 = jnp.argmax(probs, axis=-1).astype(jnp.int32)
    p1 = jnp.max(probs, axis=-1, keepdims=True)
    masked = jnp.where(ecols == i1[:, None], jnp.float32(-1.0), probs)
    i2 = jnp.argmax(masked, axis=-1).astype(jnp.int32)
    p2 = jnp.max(masked, axis=-1, keepdims=True)
    sc_ref[...] = jnp.concatenate([p1, p2], axis=1)
    idx_ref[...] = jnp.concatenate([i1[:, None], i2[:, None]], axis=1)
    onehot = ((ecols == i1[:, None]).astype(f32)
              + (ecols == i2[:, None]).astype(f32))
    cnt_part = jnp.sum(onehot, axis=0, keepdims=True)
    sp_part = jnp.sum(probs, axis=0, keepdims=True)

    @pl.when(i == 0)
    def _():
        cnt_ref[...] = jnp.zeros_like(cnt_ref)
        sp_ref[...] = jnp.zeros_like(sp_ref)

    cnt_ref[...] += cnt_part
    sp_ref[...] += sp_part

    @pl.when(i == pl.num_programs(0) - 1)
    def _():
        density = cnt_ref[...] / jnp.float32(T * K)
        meanp = sp_ref[...] / jnp.float32(T)
        loss_ref[...] = jnp.float32(E) * jnp.sum(density * meanp).reshape(1, 1)


def _moe_kernel(h2_ref, w1_ref, b1_ref, w2_ref, b2_ref, idx_ref,
                x2_ref, sc_ref, o_ref, eo_ref):
    e = pl.program_id(0)

    @pl.when(e == 0)
    def _():
        eo_ref[...] = jnp.zeros_like(eo_ref)

    for half in range(2):
        lo, hi = half * (T // 2), (half + 1) * (T // 2)
        h2 = h2_ref[lo:hi, :]
        hh = jnp.dot(h2, w1_ref[0], preferred_element_type=f32) + b1_ref[0]
        hh = jax.nn.gelu(hh).astype(bf16)
        oute = jnp.dot(hh, w2_ref[0], preferred_element_type=f32) + b2_ref[0]
        m0 = (idx_ref[lo:hi, 0:1] == e).astype(f32)
        m1 = (idx_ref[lo:hi, 1:2] == e).astype(f32)
        eo_ref[0, lo:hi, :] += (m0 * oute).astype(bf16)
        eo_ref[1, lo:hi, :] += (m1 * oute).astype(bf16)

    @pl.when(e == E - 1)
    def _():
        half_t = T // 2
        for c in range(T // BLK):
            r0, r1 = c * BLK, (c + 1) * BLK
            t0, t1 = c * (BLK // 2), (c + 1) * (BLK // 2)
            sc = sc_ref[r0:r1, :]
            a = jnp.concatenate([eo_ref[0, t0:t1][:, None, :],
                                 eo_ref[1, t0:t1][:, None, :]],
                                axis=1).reshape(BLK, D).astype(f32)
            b = jnp.concatenate([eo_ref[0, half_t + t0:half_t + t1][:, None, :],
                                 eo_ref[1, half_t + t0:half_t + t1][:, None, :]],
                                axis=1).reshape(BLK, D).astype(f32)
            o_ref[r0:r1, :] = (x2_ref[r0:r1, :] + sc[:, 0:1] * a
                               + sc[:, 1:2] * b)


def kernel(x, g_attn, wq, wk, wv, wo, g_mlp, w_router, w1, b1, w2, b2):
    x2d = x.reshape(T, D)
    wqkv = jnp.concatenate([wq, wk, wv], axis=1).astype(bf16)

    qkv = pl.pallas_call(
        _qkv_kernel,
        grid=(T // BLK,),
        in_specs=[
            pl.BlockSpec((BLK, D), lambda i: (i, 0)),
            pl.BlockSpec((1, D), lambda i: (0, 0)),
            pl.BlockSpec((D, 3 * D), lambda i: (0, 0)),
        ],
        out_specs=pl.BlockSpec((BLK, 3 * D), lambda i: (i, 0)),
        out_shape=jax.ShapeDtypeStruct((T, 3 * D), bf16),
    )(x2d, g_attn.reshape(1, D), wqkv)

    attn = pl.pallas_call(
        _attn_kernel,
        grid=(HP, S // QBLK),
        in_specs=[
            pl.BlockSpec((QBLK, 2 * Dh), lambda h, i: (i, h)),
            pl.BlockSpec((S, 2 * Dh), lambda h, i: (0, HP + h)),
            pl.BlockSpec((S, 2 * Dh), lambda h, i: (0, 2 * HP + h)),
        ],
        out_specs=pl.BlockSpec((QBLK, 2 * Dh), lambda h, i: (i, h)),
        out_shape=jax.ShapeDtypeStruct((T, D), bf16),
        scratch_shapes=[
            pltpu.VMEM((QBLK, 1), f32),
            pltpu.VMEM((QBLK, 2 * Dh), f32),
        ],
    )(qkv, qkv, qkv)

    x2, h2, sc, idx, cnt, sp, loss = pl.pallas_call(
        _router_kernel,
        grid=(T // BLK,),
        in_specs=[
            pl.BlockSpec((BLK, D), lambda i: (i, 0)),
            pl.BlockSpec((BLK, D), lambda i: (i, 0)),
            pl.BlockSpec((D, D), lambda i: (0, 0)),
            pl.BlockSpec((1, D), lambda i: (0, 0)),
            pl.BlockSpec((D, E), lambda i: (0, 0)),
        ],
        out_specs=[
            pl.BlockSpec((BLK, D), lambda i: (i, 0)),
            pl.BlockSpec((BLK, D), lambda i: (i, 0)),
            pl.BlockSpec((BLK, K), lambda i: (i, 0)),
            pl.BlockSpec((BLK, K), lambda i: (i, 0)),
            pl.BlockSpec((1, E), lambda i: (0, 0)),
            pl.BlockSpec((1, E), lambda i: (0, 0)),
            pl.BlockSpec((1, 1), lambda i: (0, 0)),
        ],
        out_shape=[
            jax.ShapeDtypeStruct((T, D), f32),
            jax.ShapeDtypeStruct((T, D), bf16),
            jax.ShapeDtypeStruct((T, K), f32),
            jax.ShapeDtypeStruct((T, K), jnp.int32),
            jax.ShapeDtypeStruct((1, E), f32),
            jax.ShapeDtypeStruct((1, E), f32),
            jax.ShapeDtypeStruct((1, 1), f32),
        ],
    )(x2d, attn, wo.astype(bf16), g_mlp.reshape(1, D), w_router)

    final = pl.pallas_call(
        _moe_kernel,
        grid=(E,),
        in_specs=[
            pl.BlockSpec((T, D), lambda e: (0, 0)),
            pl.BlockSpec((1, D, DFF), lambda e: (e, 0, 0)),
            pl.BlockSpec((1, 1, DFF), lambda e: (e, 0, 0)),
            pl.BlockSpec((1, DFF, D), lambda e: (e, 0, 0)),
            pl.BlockSpec((1, 1, D), lambda e: (e, 0, 0)),
            pl.BlockSpec((T, K), lambda e: (0, 0)),
            pl.BlockSpec((T, D), lambda e: (0, 0)),
            pl.BlockSpec((T, K), lambda e: (0, 0)),
        ],
        out_specs=pl.BlockSpec((T, D), lambda e: (0, 0)),
        out_shape=jax.ShapeDtypeStruct((T, D), f32),
        scratch_shapes=[pltpu.VMEM((K, T, D), bf16)],
        compiler_params=pltpu.CompilerParams(
            vmem_limit_bytes=100 * 1024 * 1024),
    )(h2, w1.astype(bf16), b1.reshape(E, 1, DFF), w2.astype(bf16),
      b2.reshape(E, 1, D), idx, x2, sc)

    return final.reshape(B, S, D), loss.reshape(()), cnt.reshape(E)


# speedup vs baseline: 1.7425x; 1.1672x over previous
import jax
import jax.numpy as jnp
from jax.experimental import pallas as pl
from jax.experimental.pallas import tpu as pltpu

B, S, D = 1, 2048, 768
H, Dh = 12, 64
E, K = 8, 2
DFF = 1536
EPS = 1e-6
T = B * S
BLK = 256
QBLK = 512
HP = H // 2

f32 = jnp.float32
bf16 = jnp.bfloat16


def _qkv_kernel(x_ref, g_ref, wq_ref, wk_ref, wv_ref, qkv_ref):
    x = x_ref[...]
    ms = jnp.mean(x * x, axis=-1, keepdims=True)
    h = (x * jax.lax.rsqrt(ms + EPS) * g_ref[...]).astype(bf16)
    for s, w_ref in enumerate((wq_ref, wk_ref, wv_ref)):
        w = w_ref[...].astype(bf16)
        qkv_ref[:, s * D:(s + 1) * D] = jnp.dot(
            h, w, preferred_element_type=f32).astype(bf16)


def _attn_kernel(q_ref, k_ref, v_ref, o_ref, m_ref, oacc_ref):
    i = pl.program_id(1)
    tri = (jax.lax.broadcasted_iota(jnp.int32, (QBLK, QBLK), 1)
           <= jax.lax.broadcasted_iota(jnp.int32, (QBLK, QBLK), 0))
    scale = jnp.float32(1.0) / jnp.sqrt(jnp.float32(Dh))
    ones = jnp.ones((QBLK, Dh), bf16)
    nch = S // QBLK
    for sub in range(2):
        cs = slice(sub * Dh, (sub + 1) * Dh)
        q = (q_ref[:, cs].astype(f32) * scale).astype(bf16)
        kc = k_ref[pl.ds(i * QBLK, QBLK), cs]
        vc = v_ref[pl.ds(i * QBLK, QBLK), cs]
        s = jax.lax.dot_general(q, kc, (((1,), (1,)), ((), ())),
                                preferred_element_type=f32).astype(bf16)
        s = jnp.where(tri, s, jnp.asarray(-1e9, bf16))
        mj = jnp.max(s, axis=-1, keepdims=True).astype(f32)
        m_ref[...] = mj
        p = jnp.exp(s - mj.astype(bf16))
        vext = jnp.concatenate([vc, ones], axis=1)
        oacc_ref[...] = jnp.dot(p, vext, preferred_element_type=f32)
        for j in range(nch - 1):
            @pl.when(j < i)
            def _():
                kc = k_ref[j * QBLK:(j + 1) * QBLK, cs]
                vc = v_ref[j * QBLK:(j + 1) * QBLK, cs]
                s = jax.lax.dot_general(q, kc, (((1,), (1,)), ((), ())),
                                        preferred_element_type=f32
                                        ).astype(bf16)
                mj = jnp.max(s, axis=-1, keepdims=True).astype(f32)
                mnew = jnp.maximum(m_ref[...], mj)
                corr = jnp.exp(m_ref[...] - mnew)
                p = jnp.exp(s - mnew.astype(bf16))
                vext = jnp.concatenate([vc, ones], axis=1)
                oacc_ref[...] = (oacc_ref[...] * corr
                                 + jnp.dot(p, vext,
                                           preferred_element_type=f32))
                m_ref[...] = mnew
        acc = oacc_ref[...]
        o = acc[:, :Dh] * (jnp.float32(1.0) / acc[:, Dh:Dh + 1])
        o_ref[:, cs] = o.astype(bf16)


def _router_kernel(x_ref, a_ref, wo_ref, g_ref, wr_ref,
                   x2_ref, h2_ref, sc_ref, idx_ref, cnt_ref, sp_ref, loss_ref):
    i = pl.program_id(0)
    a = jnp.dot(a_ref[...], wo_ref[...].astype(bf16),
                preferred_element_type=f32)
    x2 = x_ref[...] + a
    x2_ref[...] = x2
    ms = jnp.mean(x2 * x2, axis=-1, keepdims=True)
    h2 = x2 * jax.lax.rsqrt(ms + EPS) * g_ref[...]
    h2_ref[...] = h2.astype(bf16)
    logits = jnp.dot(h2, wr_ref[...], preferred_element_type=f32)
    m = jnp.max(logits, axis=-1, keepdims=True)
    ex = jnp.exp(logits - m)
    probs = ex / jnp.sum(ex, axis=-1, keepdims=True)
    ecols = jax.lax.broadcasted_iota(jnp.int32, (BLK, E), 1)
    i1 = jnp.argmax(probs, axis=-1).astype(jnp.int32)
    p1 = jnp.max(probs, axis=-1, keepdims=True)
    masked = jnp.where(ecols == i1[:, None], jnp.float32(-1.0), probs)
    i2 = jnp.argmax(masked, axis=-1).astype(jnp.int32)
    p2 = jnp.max(masked, axis=-1, keepdims=True)
    sc_ref[...] = jnp.concatenate([p1, p2], axis=1)
    idx_ref[...] = jnp.concatenate([i1[:, None], i2[:, None]], axis=1)
    onehot = ((ecols == i1[:, None]).astype(f32)
              + (ecols == i2[:, None]).astype(f32))
    cnt_part = jnp.sum(onehot, axis=0, keepdims=True)
    sp_part = jnp.sum(probs, axis=0, keepdims=True)

    @pl.when(i == 0)
    def _():
        cnt_ref[...] = jnp.zeros_like(cnt_ref)
        sp_ref[...] = jnp.zeros_like(sp_ref)

    cnt_ref[...] += cnt_part
    sp_ref[...] += sp_part

    @pl.when(i == pl.num_programs(0) - 1)
    def _():
        density = cnt_ref[...] / jnp.float32(T * K)
        meanp = sp_ref[...] / jnp.float32(T)
        loss_ref[...] = jnp.float32(E) * jnp.sum(density * meanp).reshape(1, 1)


def _moe_kernel(h2_ref, w1_ref, b1_ref, w2_ref, b2_ref, idx_ref,
                x2_ref, sc_ref, o_ref, eo_ref):
    e = pl.program_id(0)

    @pl.when(e == 0)
    def _():
        eo_ref[...] = jnp.zeros_like(eo_ref)

    w1b = w1_ref[0].astype(bf16)
    w2b = w2_ref[0].astype(bf16)
    nq = 4
    for quarter in range(nq):
        lo, hi = quarter * (T // nq), (quarter + 1) * (T // nq)
        h2 = h2_ref[lo:hi, :]
        hh = jnp.dot(h2, w1b, preferred_element_type=f32) + b1_ref[0]
        hh = jax.nn.gelu(hh).astype(bf16)
        oute = jnp.dot(hh, w2b, preferred_element_type=f32) + b2_ref[0]
        m0 = (idx_ref[lo:hi, 0:1] == e).astype(f32)
        m1 = (idx_ref[lo:hi, 1:2] == e).astype(f32)
        eo_ref[0, lo:hi, :] += (m0 * oute).astype(bf16)
        eo_ref[1, lo:hi, :] += (m1 * oute).astype(bf16)

    @pl.when(e == E - 1)
    def _():
        half_t = T // 2
        for c in range(T // BLK):
            r0, r1 = c * BLK, (c + 1) * BLK
            t0, t1 = c * (BLK // 2), (c + 1) * (BLK // 2)
            sc = sc_ref[r0:r1, :]
            a = jnp.concatenate([eo_ref[0, t0:t1][:, None, :],
                                 eo_ref[1, t0:t1][:, None, :]],
                                axis=1).reshape(BLK, D).astype(f32)
            b = jnp.concatenate([eo_ref[0, half_t + t0:half_t + t1][:, None, :],
                                 eo_ref[1, half_t + t0:half_t + t1][:, None, :]],
                                axis=1).reshape(BLK, D).astype(f32)
            o_ref[r0:r1, :] = (x2_ref[r0:r1, :] + sc[:, 0:1] * a
                               + sc[:, 1:2] * b)


def kernel(x, g_attn, wq, wk, wv, wo, g_mlp, w_router, w1, b1, w2, b2):
    x2d = x.reshape(T, D)

    qkv = pl.pallas_call(
        _qkv_kernel,
        grid=(T // BLK,),
        in_specs=[
            pl.BlockSpec((BLK, D), lambda i: (i, 0)),
            pl.BlockSpec((1, D), lambda i: (0, 0)),
            pl.BlockSpec((D, D), lambda i: (0, 0)),
            pl.BlockSpec((D, D), lambda i: (0, 0)),
            pl.BlockSpec((D, D), lambda i: (0, 0)),
        ],
        out_specs=pl.BlockSpec((BLK, 3 * D), lambda i: (i, 0)),
        out_shape=jax.ShapeDtypeStruct((T, 3 * D), bf16),
    )(x2d, g_attn.reshape(1, D), wq, wk, wv)

    attn = pl.pallas_call(
        _attn_kernel,
        grid=(HP, S // QBLK),
        in_specs=[
            pl.BlockSpec((QBLK, 2 * Dh), lambda h, i: (i, h)),
            pl.BlockSpec((S, 2 * Dh), lambda h, i: (0, HP + h)),
            pl.BlockSpec((S, 2 * Dh), lambda h, i: (0, 2 * HP + h)),
        ],
        out_specs=pl.BlockSpec((QBLK, 2 * Dh), lambda h, i: (i, h)),
        out_shape=jax.ShapeDtypeStruct((T, D), bf16),
        scratch_shapes=[
            pltpu.VMEM((QBLK, 1), f32),
            pltpu.VMEM((QBLK, 2 * Dh), f32),
        ],
    )(qkv, qkv, qkv)

    x2, h2, sc, idx, cnt, sp, loss = pl.pallas_call(
        _router_kernel,
        grid=(T // BLK,),
        in_specs=[
            pl.BlockSpec((BLK, D), lambda i: (i, 0)),
            pl.BlockSpec((BLK, D), lambda i: (i, 0)),
            pl.BlockSpec((D, D), lambda i: (0, 0)),
            pl.BlockSpec((1, D), lambda i: (0, 0)),
            pl.BlockSpec((D, E), lambda i: (0, 0)),
        ],
        out_specs=[
            pl.BlockSpec((BLK, D), lambda i: (i, 0)),
            pl.BlockSpec((BLK, D), lambda i: (i, 0)),
            pl.BlockSpec((BLK, K), lambda i: (i, 0)),
            pl.BlockSpec((BLK, K), lambda i: (i, 0)),
            pl.BlockSpec((1, E), lambda i: (0, 0)),
            pl.BlockSpec((1, E), lambda i: (0, 0)),
            pl.BlockSpec((1, 1), lambda i: (0, 0)),
        ],
        out_shape=[
            jax.ShapeDtypeStruct((T, D), f32),
            jax.ShapeDtypeStruct((T, D), bf16),
            jax.ShapeDtypeStruct((T, K), f32),
            jax.ShapeDtypeStruct((T, K), jnp.int32),
            jax.ShapeDtypeStruct((1, E), f32),
            jax.ShapeDtypeStruct((1, E), f32),
            jax.ShapeDtypeStruct((1, 1), f32),
        ],
    )(x2d, attn, wo, g_mlp.reshape(1, D), w_router)

    final = pl.pallas_call(
        _moe_kernel,
        grid=(E,),
        in_specs=[
            pl.BlockSpec((T, D), lambda e: (0, 0)),
            pl.BlockSpec((1, D, DFF), lambda e: (e, 0, 0)),
            pl.BlockSpec((1, 1, DFF), lambda e: (e, 0, 0)),
            pl.BlockSpec((1, DFF, D), lambda e: (e, 0, 0)),
            pl.BlockSpec((1, 1, D), lambda e: (e, 0, 0)),
            pl.BlockSpec((T, K), lambda e: (0, 0)),
            pl.BlockSpec((T, D), lambda e: (0, 0)),
            pl.BlockSpec((T, K), lambda e: (0, 0)),
        ],
        out_specs=pl.BlockSpec((T, D), lambda e: (0, 0)),
        out_shape=jax.ShapeDtypeStruct((T, D), f32),
        scratch_shapes=[pltpu.VMEM((K, T, D), bf16)],
        compiler_params=pltpu.CompilerParams(
            vmem_limit_bytes=100 * 1024 * 1024),
    )(h2, w1, b1.reshape(E, 1, DFF), w2, b2.reshape(E, 1, D), idx, x2, sc)

    return final.reshape(B, S, D), loss.reshape(()), cnt.reshape(E)


# speedup vs baseline: 1.7657x; 1.0133x over previous
import jax
import jax.numpy as jnp
from jax.experimental import pallas as pl
from jax.experimental.pallas import tpu as pltpu

B, S, D = 1, 2048, 768
H, Dh = 12, 64
E, K = 8, 2
DFF = 1536
EPS = 1e-6
T = B * S
BLK = 256
QBLK = 512
HP = H // 2

f32 = jnp.float32
bf16 = jnp.bfloat16


def _qkv_kernel(x_ref, g_ref, wq_ref, wk_ref, wv_ref, qkv_ref):
    x = x_ref[...]
    ms = jnp.mean(x * x, axis=-1, keepdims=True)
    h = (x * jax.lax.rsqrt(ms + EPS) * g_ref[...]).astype(bf16)
    for s, w_ref in enumerate((wq_ref, wk_ref, wv_ref)):
        w = w_ref[...].astype(bf16)
        qkv_ref[:, s * D:(s + 1) * D] = jnp.dot(
            h, w, preferred_element_type=f32).astype(bf16)


def _attn_kernel(q_ref, k_ref, v_ref, o_ref, m_ref, oacc_ref):
    i = pl.program_id(1)
    tri = (jax.lax.broadcasted_iota(jnp.int32, (QBLK, QBLK), 1)
           <= jax.lax.broadcasted_iota(jnp.int32, (QBLK, QBLK), 0))
    scale = jnp.float32(1.0) / jnp.sqrt(jnp.float32(Dh))
    ones = jnp.ones((QBLK, Dh), bf16)
    nch = S // QBLK
    for sub in range(2):
        cs = slice(sub * Dh, (sub + 1) * Dh)
        q = (q_ref[:, cs].astype(f32) * scale).astype(bf16)
        kc = k_ref[pl.ds(i * QBLK, QBLK), cs]
        vc = v_ref[pl.ds(i * QBLK, QBLK), cs]
        s = jax.lax.dot_general(q, kc, (((1,), (1,)), ((), ())),
                                preferred_element_type=f32).astype(bf16)
        s = jnp.where(tri, s, jnp.asarray(-1e9, bf16))
        mj = jnp.max(s, axis=-1, keepdims=True).astype(f32)
        m_ref[...] = mj
        p = jnp.exp(s - mj.astype(bf16))
        vext = jnp.concatenate([vc, ones], axis=1)
        oacc_ref[...] = jnp.dot(p, vext, preferred_element_type=f32)
        for j in range(nch - 1):
            @pl.when(j < i)
            def _():
                kc = k_ref[j * QBLK:(j + 1) * QBLK, cs]
                vc = v_ref[j * QBLK:(j + 1) * QBLK, cs]
                s = jax.lax.dot_general(q, kc, (((1,), (1,)), ((), ())),
                                        preferred_element_type=f32
                                        ).astype(bf16)
                mj = jnp.max(s, axis=-1, keepdims=True).astype(f32)
                mnew = jnp.maximum(m_ref[...], mj)
                corr = jnp.exp(m_ref[...] - mnew)
                p = jnp.exp(s - mnew.astype(bf16))
                vext = jnp.concatenate([vc, ones], axis=1)
                oacc_ref[...] = (oacc_ref[...] * corr
                                 + jnp.dot(p, vext,
                                           preferred_element_type=f32))
                m_ref[...] = mnew
        acc = oacc_ref[...]
        o = acc[:, :Dh] * (jnp.float32(1.0) / acc[:, Dh:Dh + 1])
        o_ref[:, cs] = o.astype(bf16)


def _router_kernel(x_ref, a_ref, wo_ref, g_ref, wr_ref,
                   x2_ref, h2_ref, sc_ref, idx_ref, cnt_ref, sp_ref, loss_ref):
    i = pl.program_id(0)
    a = jnp.dot(a_ref[...], wo_ref[...].astype(bf16),
                preferred_element_type=f32)
    x2 = x_ref[...] + a
    x2_ref[...] = x2
    ms = jnp.mean(x2 * x2, axis=-1, keepdims=True)
    h2 = x2 * jax.lax.rsqrt(ms + EPS) * g_ref[...]
    h2_ref[...] = h2.astype(bf16)
    logits = jnp.dot(h2, wr_ref[...], preferred_element_type=f32)
    m = jnp.max(logits, axis=-1, keepdims=True)
    ex = jnp.exp(logits - m)
    probs = ex / jnp.sum(ex, axis=-1, keepdims=True)
    ecols = jax.lax.broadcasted_iota(jnp.int32, (BLK, E), 1)
    i1 = jnp.argmax(probs, axis=-1).astype(jnp.int32)
    p1 = jnp.max(probs, axis=-1, keepdims=True)
    masked = jnp.where(ecols == i1[:, None], jnp.float32(-1.0), probs)
    i2 = jnp.argmax(masked, axis=-1).astype(jnp.int32)
    p2 = jnp.max(masked, axis=-1, keepdims=True)
    sc_ref[...] = jnp.concatenate([p1, p2], axis=1)
    idx_ref[...] = jnp.concatenate([i1[:, None], i2[:, None]], axis=1)
    onehot = ((ecols == i1[:, None]).astype(f32)
              + (ecols == i2[:, None]).astype(f32))
    cnt_part = jnp.sum(onehot, axis=0, keepdims=True)
    sp_part = jnp.sum(probs, axis=0, keepdims=True)

    @pl.when(i == 0)
    def _():
        cnt_ref[...] = jnp.zeros_like(cnt_ref)
        sp_ref[...] = jnp.zeros_like(sp_ref)

    cnt_ref[...] += cnt_part
    sp_ref[...] += sp_part

    @pl.when(i == pl.num_programs(0) - 1)
    def _():
        density = cnt_ref[...] / jnp.float32(T * K)
        meanp = sp_ref[...] / jnp.float32(T)
        loss_ref[...] = jnp.float32(E) * jnp.sum(density * meanp).reshape(1, 1)


def _moe_kernel(h2_ref, w1_ref, b1_ref, w2_ref, b2_ref, idx_ref,
                x2_ref, sc_ref, o_ref, eo_ref):
    e = pl.program_id(0)
    w1b = w1_ref[0].astype(bf16)
    w2b = w2_ref[0].astype(bf16)
    nq = 4
    for quarter in range(nq):
        lo, hi = quarter * (T // nq), (quarter + 1) * (T // nq)
        h2 = h2_ref[lo:hi, :]
        hh = (jnp.dot(h2, w1b, preferred_element_type=f32)
              + b1_ref[0]).astype(bf16)
        hh = jax.nn.gelu(hh)
        oute = (jnp.dot(hh, w2b, preferred_element_type=f32)
                + b2_ref[0]).astype(bf16)
        m0 = idx_ref[lo:hi, 0:1] == e
        m1 = idx_ref[lo:hi, 1:2] == e
        eo_ref[0, lo:hi, :] = jnp.where(m0, oute, eo_ref[0, lo:hi, :])
        eo_ref[1, lo:hi, :] = jnp.where(m1, oute, eo_ref[1, lo:hi, :])

    @pl.when(e == E - 1)
    def _():
        half_t = T // 2
        for c in range(T // BLK):
            r0, r1 = c * BLK, (c + 1) * BLK
            t0, t1 = c * (BLK // 2), (c + 1) * (BLK // 2)
            sc = sc_ref[r0:r1, :]
            a = jnp.concatenate([eo_ref[0, t0:t1][:, None, :],
                                 eo_ref[1, t0:t1][:, None, :]],
                                axis=1).reshape(BLK, D).astype(f32)
            b = jnp.concatenate([eo_ref[0, half_t + t0:half_t + t1][:, None, :],
                                 eo_ref[1, half_t + t0:half_t + t1][:, None, :]],
                                axis=1).reshape(BLK, D).astype(f32)
            o_ref[r0:r1, :] = (x2_ref[r0:r1, :] + sc[:, 0:1] * a
                               + sc[:, 1:2] * b)


def kernel(x, g_attn, wq, wk, wv, wo, g_mlp, w_router, w1, b1, w2, b2):
    x2d = x.reshape(T, D)

    qkv = pl.pallas_call(
        _qkv_kernel,
        grid=(T // BLK,),
        in_specs=[
            pl.BlockSpec((BLK, D), lambda i: (i, 0)),
            pl.BlockSpec((1, D), lambda i: (0, 0)),
            pl.BlockSpec((D, D), lambda i: (0, 0)),
            pl.BlockSpec((D, D), lambda i: (0, 0)),
            pl.BlockSpec((D, D), lambda i: (0, 0)),
        ],
        out_specs=pl.BlockSpec((BLK, 3 * D), lambda i: (i, 0)),
        out_shape=jax.ShapeDtypeStruct((T, 3 * D), bf16),
    )(x2d, g_attn.reshape(1, D), wq, wk, wv)

    attn = pl.pallas_call(
        _attn_kernel,
        grid=(HP, S // QBLK),
        in_specs=[
            pl.BlockSpec((QBLK, 2 * Dh), lambda h, i: (i, h)),
            pl.BlockSpec((S, 2 * Dh), lambda h, i: (0, HP + h)),
            pl.BlockSpec((S, 2 * Dh), lambda h, i: (0, 2 * HP + h)),
        ],
        out_specs=pl.BlockSpec((QBLK, 2 * Dh), lambda h, i: (i, h)),
        out_shape=jax.ShapeDtypeStruct((T, D), bf16),
        scratch_shapes=[
            pltpu.VMEM((QBLK, 1), f32),
            pltpu.VMEM((QBLK, 2 * Dh), f32),
        ],
    )(qkv, qkv, qkv)

    x2, h2, sc, idx, cnt, sp, loss = pl.pallas_call(
        _router_kernel,
        grid=(T // BLK,),
        in_specs=[
            pl.BlockSpec((BLK, D), lambda i: (i, 0)),
            pl.BlockSpec((BLK, D), lambda i: (i, 0)),
            pl.BlockSpec((D, D), lambda i: (0, 0)),
            pl.BlockSpec((1, D), lambda i: (0, 0)),
            pl.BlockSpec((D, E), lambda i: (0, 0)),
        ],
        out_specs=[
            pl.BlockSpec((BLK, D), lambda i: (i, 0)),
            pl.BlockSpec((BLK, D), lambda i: (i, 0)),
            pl.BlockSpec((BLK, K), lambda i: (i, 0)),
            pl.BlockSpec((BLK, K), lambda i: (i, 0)),
            pl.BlockSpec((1, E), lambda i: (0, 0)),
            pl.BlockSpec((1, E), lambda i: (0, 0)),
            pl.BlockSpec((1, 1), lambda i: (0, 0)),
        ],
        out_shape=[
            jax.ShapeDtypeStruct((T, D), f32),
            jax.ShapeDtypeStruct((T, D), bf16),
            jax.ShapeDtypeStruct((T, K), f32),
            jax.ShapeDtypeStruct((T, K), jnp.int32),
            jax.ShapeDtypeStruct((1, E), f32),
            jax.ShapeDtypeStruct((1, E), f32),
            jax.ShapeDtypeStruct((1, 1), f32),
        ],
    )(x2d, attn, wo, g_mlp.reshape(1, D), w_router)

    final = pl.pallas_call(
        _moe_kernel,
        grid=(E,),
        in_specs=[
            pl.BlockSpec((T, D), lambda e: (0, 0)),
            pl.BlockSpec((1, D, DFF), lambda e: (e, 0, 0)),
            pl.BlockSpec((1, 1, DFF), lambda e: (e, 0, 0)),
            pl.BlockSpec((1, DFF, D), lambda e: (e, 0, 0)),
            pl.BlockSpec((1, 1, D), lambda e: (e, 0, 0)),
            pl.BlockSpec((T, K), lambda e: (0, 0)),
            pl.BlockSpec((T, D), lambda e: (0, 0)),
            pl.BlockSpec((T, K), lambda e: (0, 0)),
        ],
        out_specs=pl.BlockSpec((T, D), lambda e: (0, 0)),
        out_shape=jax.ShapeDtypeStruct((T, D), f32),
        scratch_shapes=[pltpu.VMEM((K, T, D), bf16)],
        compiler_params=pltpu.CompilerParams(
            vmem_limit_bytes=100 * 1024 * 1024),
    )(h2, w1, b1.reshape(E, 1, DFF), w2, b2.reshape(E, 1, D), idx, x2, sc)

    return final.reshape(B, S, D), loss.reshape(()), cnt.reshape(E)
